# baseline (speedup 1.0000x reference)
import functools

import jax
import jax.numpy as jnp
from jax.experimental import pallas as pl
from jax.experimental.pallas import tpu as pltpu
from jax.experimental.pallas import tpu_sc as plsc

E = 64
K = 2
D = 1024
DFF = 2048
B = 4
S = 2048
T = B * S
TK = T * K

TILE_M = 128
NT = 192
NP = NT * TILE_M
RB = 512
QD = 256
QW = 128
NWORKERS = 32


def _router_body(xb_ref, wr_ref, g1_ref, g2_ref, i1_ref, i2_ref):
    xb = xb_ref[...]
    wr = wr_ref[...]
    logits = jax.lax.dot_general(xb, wr, (((1,), (1,)), ((), ())),
                                 preferred_element_type=jnp.float32)
    iota = jax.lax.broadcasted_iota(jnp.int32, logits.shape, 1)
    m1 = jnp.max(logits, axis=1, keepdims=True)
    i1 = jnp.min(jnp.where(logits == m1, iota, E), axis=1, keepdims=True)
    masked = jnp.where(iota == i1, -jnp.inf, logits)
    m2 = jnp.max(masked, axis=1, keepdims=True)
    i2 = jnp.min(jnp.where(masked == m2, iota, E), axis=1, keepdims=True)
    e2 = jnp.exp(m2 - m1)
    denom = 1.0 + e2
    g1_ref[0, 0, :] = (1.0 / denom)[:, 0]
    g2_ref[0, 0, :] = (e2 / denom)[:, 0]
    i1_ref[0, 0, :] = i1[:, 0]
    i2_ref[0, 0, :] = i2[:, 0]


def _router(flat, W_router):
    nb = T // RB
    outs = pl.pallas_call(
        _router_body,
        grid=(nb,),
        in_specs=[
            pl.BlockSpec((RB, D), lambda i: (i, 0)),
            pl.BlockSpec((E, D), lambda i: (0, 0)),
        ],
        out_specs=[pl.BlockSpec((1, 1, RB), lambda i: (i, 0, 0))] * 4,
        out_shape=[
            jax.ShapeDtypeStruct((nb, 1, RB), jnp.float32),
            jax.ShapeDtypeStruct((nb, 1, RB), jnp.float32),
            jax.ShapeDtypeStruct((nb, 1, RB), jnp.int32),
            jax.ShapeDtypeStruct((nb, 1, RB), jnp.int32),
        ],
    )(flat, W_router)
    g1, g2, i1, i2 = outs
    return g1.reshape(T), g2.reshape(T), i1.reshape(T), i2.reshape(T)


def _bookkeeping(g1, g2, i1, i2):
    e_flat = jnp.stack([i1, i2], axis=1).reshape(TK)
    g_flat = jnp.stack([g1, g2], axis=1).reshape(TK)
    order = jnp.argsort(e_flat).astype(jnp.int32)
    sorted_e = e_flat[order]
    counts = jnp.zeros((E,), jnp.int32).at[e_flat].add(1)
    tiles_per_e = (counts + TILE_M - 1) // TILE_M
    tile_end = jnp.cumsum(tiles_per_e).astype(jnp.int32)
    row_start = (tile_end - tiles_per_e) * TILE_M
    offs = jnp.cumsum(counts).astype(jnp.int32) - counts
    dest = row_start[sorted_e] + (jnp.arange(TK, dtype=jnp.int32) - offs[sorted_e])
    gidx = jnp.zeros((NP,), jnp.int32).at[dest].set(order // K)
    gates_pad = jnp.zeros((NP,), jnp.float32).at[dest].set(g_flat[order])
    pos = jnp.zeros((TK,), jnp.int32).at[order].set(dest)
    pos0 = pos[0::2]
    pos1 = pos[1::2]
    tid = jnp.arange(NT, dtype=jnp.int32)
    total = tile_end[E - 1]
    eid_raw = jnp.searchsorted(tile_end, tid, side="right").astype(jnp.int32)
    last_e = jnp.searchsorted(tile_end, total - 1, side="right").astype(jnp.int32)
    eid = jnp.where(tid < total, jnp.minimum(eid_raw, E - 1), last_e)
    act = (tid < total).astype(jnp.int32)
    return gidx, gates_pad, pos0, pos1, eid, act


def _expand_quarters(idx):
    return (4 * idx[:, None] + jnp.arange(4, dtype=jnp.int32)[None, :]).reshape(-1)


def _sc_gather(flat, gidx):
    flat4 = flat.reshape(4 * T, QD)
    gidx4 = _expand_quarters(gidx).reshape(1, 4 * NP)
    mesh = plsc.VectorSubcoreMesh(core_axis_name="core", subcore_axis_name="subcore")
    nsteps = (4 * NP) // QW
    spw = nsteps // NWORKERS

    @functools.partial(
        pl.kernel,
        out_type=jax.ShapeDtypeStruct((4 * NP, QD), jnp.float32),
        mesh=mesh,
    )
    def k(x_hbm, i_hbm, o_hbm):
        def body(i_vmem, o_vmem):
            pltpu.sync_copy(x_hbm.at[i_vmem.at[0]], o_vmem)

        pltpu.emit_pipeline(
            body,
            grid=(NWORKERS, spw),
            in_specs=[pl.BlockSpec((1, QW), lambda w, j, _spw=spw: (0, w * _spw + j))],
            out_specs=[pl.BlockSpec((QW, QD), lambda w, j, _spw=spw: (w * _spw + j, 0))],
            core_axis_name=("core", "subcore"),
            dimension_semantics=(pltpu.PARALLEL, pltpu.ARBITRARY),
        )(i_hbm, o_hbm)

    return k(flat4, gidx4).reshape(NP, D)


def _ffn_body(eid_ref, act_ref, xb_ref, wg_ref, wu_ref, wd_ref, bd_ref, gate_ref,
              o_ref):
    t = pl.program_id(0)

    @pl.when(act_ref[t] == 1)
    def _():
        xb = xb_ref[...].astype(jnp.bfloat16)
        hg = jax.lax.dot_general(xb, wg_ref[0], (((1,), (1,)), ((), ())),
                                 preferred_element_type=jnp.float32)
        hu = jax.lax.dot_general(xb, wu_ref[0], (((1,), (1,)), ((), ())),
                                 preferred_element_type=jnp.float32)
        h = ((hg * jax.lax.logistic(hg)) * hu).astype(jnp.bfloat16)
        o = jax.lax.dot_general(h, wd_ref[0], (((1,), (1,)), ((), ())),
                                preferred_element_type=jnp.float32)
        o = o + bd_ref[0]
        o_ref[...] = gate_ref[0, 0, :][:, None] * o


def _grouped_ffn(X_s, Wg, Wu, Wd, bd, gates_pad, eid, act):
    gates3 = gates_pad.reshape(NT, 1, TILE_M)
    bd3 = bd.reshape(E, 1, D)
    grid_spec = pltpu.PrefetchScalarGridSpec(
        num_scalar_prefetch=2,
        grid=(NT,),
        in_specs=[
            pl.BlockSpec((TILE_M, D), lambda t, eid, act: (t, 0)),
            pl.BlockSpec((1, DFF, D), lambda t, eid, act: (eid[t], 0, 0)),
            pl.BlockSpec((1, DFF, D), lambda t, eid, act: (eid[t], 0, 0)),
            pl.BlockSpec((1, D, DFF), lambda t, eid, act: (eid[t], 0, 0)),
            pl.BlockSpec((1, 1, D), lambda t, eid, act: (eid[t], 0, 0)),
            pl.BlockSpec((1, 1, TILE_M), lambda t, eid, act: (t, 0, 0)),
        ],
        out_specs=pl.BlockSpec((TILE_M, D), lambda t, eid, act: (t, 0)),
    )
    return pl.pallas_call(
        _ffn_body,
        grid_spec=grid_spec,
        out_shape=jax.ShapeDtypeStruct((NP, D), jnp.float32),
    )(eid, act, X_s, Wg, Wu, Wd, bd3, gates3)


def _sc_combine(O, pos0, pos1):
    O4 = O.reshape(4 * NP, QD)
    q0 = _expand_quarters(pos0).reshape(1, 4 * T)
    q1 = _expand_quarters(pos1).reshape(1, 4 * T)
    mesh = plsc.VectorSubcoreMesh(core_axis_name="core", subcore_axis_name="subcore")
    nsteps = (4 * T) // QW
    spw = nsteps // NWORKERS

    @functools.partial(
        pl.kernel,
        out_type=jax.ShapeDtypeStruct((4 * T, QD), jnp.float32),
        mesh=mesh,
        scratch_types=[pltpu.VMEM((QW, QD), jnp.float32)],
    )
    def k(o_hbm, p0_hbm, p1_hbm, out_hbm, scr):
        def body(p0_vmem, p1_vmem, out_vmem):
            pltpu.sync_copy(o_hbm.at[p0_vmem.at[0]], out_vmem)
            pltpu.sync_copy(o_hbm.at[p1_vmem.at[0]], scr)

            @pl.loop(0, QW)
            def _(r):
                for c in range(0, QD, 16):
                    slc = (pl.ds(r, 1), pl.ds(c, 16))
                    out_vmem.at[*slc][...] = out_vmem.at[*slc][...] + scr.at[*slc][...]

        pltpu.emit_pipeline(
            body,
            grid=(NWORKERS, spw),
            in_specs=[
                pl.BlockSpec((1, QW), lambda w, j, _spw=spw: (0, w * _spw + j)),
                pl.BlockSpec((1, QW), lambda w, j, _spw=spw: (0, w * _spw + j)),
            ],
            out_specs=[pl.BlockSpec((QW, QD), lambda w, j, _spw=spw: (w * _spw + j, 0))],
            core_axis_name=("core", "subcore"),
            dimension_semantics=(pltpu.PARALLEL, pltpu.ARBITRARY),
        )(p0_hbm, p1_hbm, out_hbm)

    return k(O4, q0, q1).reshape(T, D)


def kernel(x, W_router, Wg, Wu, Wd, bd):
    flat = x.reshape(T, D)
    g1, g2, i1, i2 = _router(flat, W_router)
    gidx, gates_pad, pos0, pos1, eid, act = _bookkeeping(g1, g2, i1, i2)
    X_s = _sc_gather(flat, gidx)
    O = _grouped_ffn(X_s, Wg.astype(jnp.bfloat16), Wu.astype(jnp.bfloat16),
                     Wd.astype(jnp.bfloat16), bd, gates_pad, eid, act)
    res = _sc_combine(O, pos0, pos1)
    return res.reshape(x.shape)


# speedup vs baseline: 1.0642x; 1.0642x over previous
import functools

import jax
import jax.numpy as jnp
from jax.experimental import pallas as pl
from jax.experimental.pallas import tpu as pltpu
from jax.experimental.pallas import tpu_sc as plsc

E = 64
K = 2
D = 1024
DFF = 2048
B = 4
S = 2048
T = B * S
TK = T * K

TILE_M = 128
NT = 192
NP = NT * TILE_M
RB = 512
QD = 256
QW = 128
NWORKERS = 32


def _router_body(xb_ref, wr_ref, g1_ref, g2_ref, i1_ref, i2_ref):
    xb = xb_ref[...]
    wr = wr_ref[...]
    logits = jax.lax.dot_general(xb, wr, (((1,), (1,)), ((), ())),
                                 preferred_element_type=jnp.float32)
    iota = jax.lax.broadcasted_iota(jnp.int32, logits.shape, 1)
    m1 = jnp.max(logits, axis=1, keepdims=True)
    i1 = jnp.min(jnp.where(logits == m1, iota, E), axis=1, keepdims=True)
    masked = jnp.where(iota == i1, -jnp.inf, logits)
    m2 = jnp.max(masked, axis=1, keepdims=True)
    i2 = jnp.min(jnp.where(masked == m2, iota, E), axis=1, keepdims=True)
    e2 = jnp.exp(m2 - m1)
    denom = 1.0 + e2
    g1_ref[0, 0, :] = (1.0 / denom)[:, 0]
    g2_ref[0, 0, :] = (e2 / denom)[:, 0]
    i1_ref[0, 0, :] = i1[:, 0]
    i2_ref[0, 0, :] = i2[:, 0]


def _router(flat, W_router):
    nb = T // RB
    outs = pl.pallas_call(
        _router_body,
        grid=(nb,),
        in_specs=[
            pl.BlockSpec((RB, D), lambda i: (i, 0)),
            pl.BlockSpec((E, D), lambda i: (0, 0)),
        ],
        out_specs=[pl.BlockSpec((1, 1, RB), lambda i: (i, 0, 0))] * 4,
        out_shape=[
            jax.ShapeDtypeStruct((nb, 1, RB), jnp.float32),
            jax.ShapeDtypeStruct((nb, 1, RB), jnp.float32),
            jax.ShapeDtypeStruct((nb, 1, RB), jnp.int32),
            jax.ShapeDtypeStruct((nb, 1, RB), jnp.int32),
        ],
    )(flat, W_router)
    g1, g2, i1, i2 = outs
    return g1.reshape(T), g2.reshape(T), i1.reshape(T), i2.reshape(T)


def _bookkeeping(g1, g2, i1, i2):
    e_flat = jnp.stack([i1, i2], axis=1).reshape(TK)
    g_flat = jnp.stack([g1, g2], axis=1).reshape(TK)
    order = jnp.argsort(e_flat).astype(jnp.int32)
    sorted_e = e_flat[order]
    counts = jnp.zeros((E,), jnp.int32).at[e_flat].add(1)
    tiles_per_e = (counts + TILE_M - 1) // TILE_M
    tile_end = jnp.cumsum(tiles_per_e).astype(jnp.int32)
    row_start = (tile_end - tiles_per_e) * TILE_M
    offs = jnp.cumsum(counts).astype(jnp.int32) - counts
    dest = row_start[sorted_e] + (jnp.arange(TK, dtype=jnp.int32) - offs[sorted_e])
    gidx = jnp.zeros((NP,), jnp.int32).at[dest].set(order // K)
    gates_pad = jnp.zeros((NP,), jnp.float32).at[dest].set(g_flat[order])
    pos = jnp.zeros((TK,), jnp.int32).at[order].set(dest)
    pos0 = pos[0::2]
    pos1 = pos[1::2]
    tid = jnp.arange(NT, dtype=jnp.int32)
    total = tile_end[E - 1]
    eid_raw = jnp.searchsorted(tile_end, tid, side="right").astype(jnp.int32)
    last_e = jnp.searchsorted(tile_end, total - 1, side="right").astype(jnp.int32)
    eid = jnp.where(tid < total, jnp.minimum(eid_raw, E - 1), last_e)
    act = (tid < total).astype(jnp.int32)
    return gidx, gates_pad, pos0, pos1, eid, act


def _expand_quarters(idx):
    return (4 * idx[:, None] + jnp.arange(4, dtype=jnp.int32)[None, :]).reshape(-1)


def _expand_halves(idx):
    return (2 * idx[:, None] + jnp.arange(2, dtype=jnp.int32)[None, :]).reshape(-1)


def _sc_gather(flat16, gidx):
    flat_i = jax.lax.bitcast_convert_type(
        flat16.reshape(T, D // 2, 2), jnp.int32).reshape(2 * T, QD)
    hidx = _expand_halves(gidx).reshape(1, 2 * NP)
    mesh = plsc.VectorSubcoreMesh(core_axis_name="core", subcore_axis_name="subcore")
    nsteps = (2 * NP) // QW
    spw = nsteps // NWORKERS

    @functools.partial(
        pl.kernel,
        out_type=jax.ShapeDtypeStruct((2 * NP, QD), jnp.int32),
        mesh=mesh,
    )
    def k(x_hbm, i_hbm, o_hbm):
        def body(i_vmem, o_vmem):
            pltpu.sync_copy(x_hbm.at[i_vmem.at[0]], o_vmem)

        pltpu.emit_pipeline(
            body,
            grid=(NWORKERS, spw),
            in_specs=[pl.BlockSpec((1, QW), lambda w, j, _spw=spw: (0, w * _spw + j))],
            out_specs=[pl.BlockSpec((QW, QD), lambda w, j, _spw=spw: (w * _spw + j, 0))],
            core_axis_name=("core", "subcore"),
            dimension_semantics=(pltpu.PARALLEL, pltpu.ARBITRARY),
        )(i_hbm, o_hbm)

    out_i = k(flat_i, hidx)
    return jax.lax.bitcast_convert_type(
        out_i.reshape(NP, D // 2), jnp.bfloat16).reshape(NP, D)


def _ffn_body(eid_ref, act_ref, xb_ref, wg_ref, wu_ref, wd_ref, bd_ref, gate_ref,
              o_ref):
    t = pl.program_id(0)

    @pl.when(act_ref[t] == 1)
    def _():
        xb = xb_ref[...].astype(jnp.float32)
        hg = jax.lax.dot_general(xb, wg_ref[0], (((1,), (1,)), ((), ())),
                                 preferred_element_type=jnp.float32)
        hu = jax.lax.dot_general(xb, wu_ref[0], (((1,), (1,)), ((), ())),
                                 preferred_element_type=jnp.float32)
        h = (hg * jax.lax.logistic(hg)) * hu
        o = jax.lax.dot_general(h, wd_ref[0], (((1,), (1,)), ((), ())),
                                preferred_element_type=jnp.float32)
        o = o + bd_ref[0]
        o_ref[...] = gate_ref[0, 0, :][:, None] * o


def _grouped_ffn(X_s, Wg, Wu, Wd, bd, gates_pad, eid, act):
    gates3 = gates_pad.reshape(NT, 1, TILE_M)
    bd3 = bd.reshape(E, 1, D)
    grid_spec = pltpu.PrefetchScalarGridSpec(
        num_scalar_prefetch=2,
        grid=(NT,),
        in_specs=[
            pl.BlockSpec((TILE_M, D), lambda t, eid, act: (t, 0)),
            pl.BlockSpec((1, DFF, D), lambda t, eid, act: (eid[t], 0, 0)),
            pl.BlockSpec((1, DFF, D), lambda t, eid, act: (eid[t], 0, 0)),
            pl.BlockSpec((1, D, DFF), lambda t, eid, act: (eid[t], 0, 0)),
            pl.BlockSpec((1, 1, D), lambda t, eid, act: (eid[t], 0, 0)),
            pl.BlockSpec((1, 1, TILE_M), lambda t, eid, act: (t, 0, 0)),
        ],
        out_specs=pl.BlockSpec((TILE_M, D), lambda t, eid, act: (t, 0)),
    )
    return pl.pallas_call(
        _ffn_body,
        grid_spec=grid_spec,
        out_shape=jax.ShapeDtypeStruct((NP, D), jnp.float32),
    )(eid, act, X_s, Wg, Wu, Wd, bd3, gates3)


def _sc_combine(O, pos0, pos1):
    O4 = O.reshape(4 * NP, QD)
    q0 = _expand_quarters(pos0).reshape(1, 4 * T)
    q1 = _expand_quarters(pos1).reshape(1, 4 * T)
    mesh = plsc.VectorSubcoreMesh(core_axis_name="core", subcore_axis_name="subcore")
    nsteps = (4 * T) // QW
    spw = nsteps // NWORKERS

    @functools.partial(
        pl.kernel,
        out_type=jax.ShapeDtypeStruct((4 * T, QD), jnp.float32),
        mesh=mesh,
        scratch_types=[pltpu.VMEM((QW, QD), jnp.float32)],
    )
    def k(o_hbm, p0_hbm, p1_hbm, out_hbm, scr):
        def body(p0_vmem, p1_vmem, out_vmem):
            pltpu.sync_copy(o_hbm.at[p0_vmem.at[0]], out_vmem)
            pltpu.sync_copy(o_hbm.at[p1_vmem.at[0]], scr)

            @pl.loop(0, QW)
            def _(r):
                for c in range(0, QD, 16):
                    slc = (pl.ds(r, 1), pl.ds(c, 16))
                    out_vmem.at[*slc][...] = out_vmem.at[*slc][...] + scr.at[*slc][...]

        pltpu.emit_pipeline(
            body,
            grid=(NWORKERS, spw),
            in_specs=[
                pl.BlockSpec((1, QW), lambda w, j, _spw=spw: (0, w * _spw + j)),
                pl.BlockSpec((1, QW), lambda w, j, _spw=spw: (0, w * _spw + j)),
            ],
            out_specs=[pl.BlockSpec((QW, QD), lambda w, j, _spw=spw: (w * _spw + j, 0))],
            core_axis_name=("core", "subcore"),
            dimension_semantics=(pltpu.PARALLEL, pltpu.ARBITRARY),
        )(p0_hbm, p1_hbm, out_hbm)

    return k(O4, q0, q1).reshape(T, D)


def kernel(x, W_router, Wg, Wu, Wd, bd):
    flat = x.reshape(T, D)
    g1, g2, i1, i2 = _router(flat, W_router)
    gidx, gates_pad, pos0, pos1, eid, act = _bookkeeping(g1, g2, i1, i2)
    X_s = _sc_gather(flat.astype(jnp.bfloat16), gidx)
    O = _grouped_ffn(X_s, Wg, Wu, Wd, bd, gates_pad, eid, act)
    res = _sc_combine(O, pos0, pos1)
    return res.reshape(x.shape)


# speedup vs baseline: 1.3920x; 1.3080x over previous
import functools

import jax
import jax.numpy as jnp
from jax.experimental import pallas as pl
from jax.experimental.pallas import tpu as pltpu
from jax.experimental.pallas import tpu_sc as plsc

E = 64
K = 2
D = 1024
DFF = 2048
B = 4
S = 2048
T = B * S
TK = T * K

TILE_M = 128
NT = 192
NP = NT * TILE_M
RB = 512
QD = 256
QW = 128
NWORKERS = 32


def _router_body(xb_ref, wr_ref, g1_ref, g2_ref, i1_ref, i2_ref):
    xb = xb_ref[...]
    wr = wr_ref[...]
    logits = jax.lax.dot_general(xb, wr, (((1,), (1,)), ((), ())),
                                 preferred_element_type=jnp.float32)
    iota = jax.lax.broadcasted_iota(jnp.int32, logits.shape, 1)
    m1 = jnp.max(logits, axis=1, keepdims=True)
    i1 = jnp.min(jnp.where(logits == m1, iota, E), axis=1, keepdims=True)
    masked = jnp.where(iota == i1, -jnp.inf, logits)
    m2 = jnp.max(masked, axis=1, keepdims=True)
    i2 = jnp.min(jnp.where(masked == m2, iota, E), axis=1, keepdims=True)
    e2 = jnp.exp(m2 - m1)
    denom = 1.0 + e2
    g1_ref[0, 0, :] = (1.0 / denom)[:, 0]
    g2_ref[0, 0, :] = (e2 / denom)[:, 0]
    i1_ref[0, 0, :] = i1[:, 0]
    i2_ref[0, 0, :] = i2[:, 0]


def _router(flat, W_router):
    nb = T // RB
    outs = pl.pallas_call(
        _router_body,
        grid=(nb,),
        in_specs=[
            pl.BlockSpec((RB, D), lambda i: (i, 0)),
            pl.BlockSpec((E, D), lambda i: (0, 0)),
        ],
        out_specs=[pl.BlockSpec((1, 1, RB), lambda i: (i, 0, 0))] * 4,
        out_shape=[
            jax.ShapeDtypeStruct((nb, 1, RB), jnp.float32),
            jax.ShapeDtypeStruct((nb, 1, RB), jnp.float32),
            jax.ShapeDtypeStruct((nb, 1, RB), jnp.int32),
            jax.ShapeDtypeStruct((nb, 1, RB), jnp.int32),
        ],
    )(flat, W_router)
    g1, g2, i1, i2 = outs
    return g1.reshape(T), g2.reshape(T), i1.reshape(T), i2.reshape(T)


def _bookkeeping(g1, g2, i1, i2):
    e_flat = jnp.stack([i1, i2], axis=1).reshape(TK)
    g_flat = jnp.stack([g1, g2], axis=1).reshape(TK)
    oh = (e_flat[:, None] == jnp.arange(E, dtype=jnp.int32)[None, :]
          ).astype(jnp.float32)
    NB = 128
    RW = TK // NB
    oh3 = oh.reshape(NB, RW, E)
    tri_r = (jnp.arange(RW)[:, None] > jnp.arange(RW)[None, :]).astype(jnp.float32)
    intra = jnp.einsum('rs,bse->bre', tri_r, oh3)
    bsum = oh3.sum(axis=1)
    tri_b = (jnp.arange(NB)[:, None] > jnp.arange(NB)[None, :]).astype(jnp.float32)
    pre = jnp.einsum('bc,ce->be', tri_b, bsum)
    rank_f = ((intra + pre[:, None, :]).reshape(TK, E) * oh).sum(axis=1)
    counts = bsum.sum(axis=0).astype(jnp.int32)
    tiles_per_e = (counts + TILE_M - 1) // TILE_M
    tile_end = jnp.cumsum(tiles_per_e).astype(jnp.int32)
    row_start = (tile_end - tiles_per_e) * TILE_M
    start_f = jnp.einsum('te,e->t', oh, row_start.astype(jnp.float32))
    dest = (start_f + rank_f).astype(jnp.int32)
    tok = jnp.arange(TK, dtype=jnp.int32) // K
    gidx = jnp.zeros((NP,), jnp.int32).at[dest].set(tok)
    gates_pad = jnp.zeros((NP,), jnp.float32).at[dest].set(g_flat)
    pos0 = dest[0::2]
    pos1 = dest[1::2]
    tid = jnp.arange(NT, dtype=jnp.int32)
    total = tile_end[E - 1]
    eid_raw = jnp.searchsorted(tile_end, tid, side="right").astype(jnp.int32)
    last_e = jnp.searchsorted(tile_end, total - 1, side="right").astype(jnp.int32)
    eid = jnp.where(tid < total, jnp.minimum(eid_raw, E - 1), last_e)
    act = (tid < total).astype(jnp.int32)
    return gidx, gates_pad, pos0, pos1, eid, act


def _expand_quarters(idx):
    return (4 * idx[:, None] + jnp.arange(4, dtype=jnp.int32)[None, :]).reshape(-1)


def _sc_gather(flat, gidx):
    flat4 = flat.reshape(4 * T, QD)
    gidx4 = _expand_quarters(gidx).reshape(1, 4 * NP)
    mesh = plsc.VectorSubcoreMesh(core_axis_name="core", subcore_axis_name="subcore")
    nsteps = (4 * NP) // QW
    spw = nsteps // NWORKERS

    @functools.partial(
        pl.kernel,
        out_type=jax.ShapeDtypeStruct((4 * NP, QD), jnp.float32),
        mesh=mesh,
    )
    def k(x_hbm, i_hbm, o_hbm):
        def body(i_vmem, o_vmem):
            pltpu.sync_copy(x_hbm.at[i_vmem.at[0]], o_vmem)

        pltpu.emit_pipeline(
            body,
            grid=(NWORKERS, spw),
            in_specs=[pl.BlockSpec((1, QW), lambda w, j, _spw=spw: (0, w * _spw + j))],
            out_specs=[pl.BlockSpec((QW, QD), lambda w, j, _spw=spw: (w * _spw + j, 0))],
            core_axis_name=("core", "subcore"),
            dimension_semantics=(pltpu.PARALLEL, pltpu.ARBITRARY),
        )(i_hbm, o_hbm)

    return k(flat4, gidx4).reshape(NP, D)


def _ffn_body(eid_ref, act_ref, xb_ref, wg_ref, wu_ref, wd_ref, bd_ref, gate_ref,
              o_ref):
    t = pl.program_id(0)

    @pl.when(act_ref[t] == 1)
    def _():
        xb = xb_ref[...]
        hg = jax.lax.dot_general(xb, wg_ref[0], (((1,), (1,)), ((), ())),
                                 preferred_element_type=jnp.float32)
        hu = jax.lax.dot_general(xb, wu_ref[0], (((1,), (1,)), ((), ())),
                                 preferred_element_type=jnp.float32)
        h = (hg * jax.lax.logistic(hg)) * hu
        o = jax.lax.dot_general(h, wd_ref[0], (((1,), (1,)), ((), ())),
                                preferred_element_type=jnp.float32)
        o = o + bd_ref[0]
        o_ref[...] = gate_ref[0, 0, :][:, None] * o


def _grouped_ffn(X_s, Wg, Wu, Wd, bd, gates_pad, eid, act):
    gates3 = gates_pad.reshape(NT, 1, TILE_M)
    bd3 = bd.reshape(E, 1, D)
    grid_spec = pltpu.PrefetchScalarGridSpec(
        num_scalar_prefetch=2,
        grid=(NT,),
        in_specs=[
            pl.BlockSpec((TILE_M, D), lambda t, eid, act: (t, 0)),
            pl.BlockSpec((1, DFF, D), lambda t, eid, act: (eid[t], 0, 0)),
            pl.BlockSpec((1, DFF, D), lambda t, eid, act: (eid[t], 0, 0)),
            pl.BlockSpec((1, D, DFF), lambda t, eid, act: (eid[t], 0, 0)),
            pl.BlockSpec((1, 1, D), lambda t, eid, act: (eid[t], 0, 0)),
            pl.BlockSpec((1, 1, TILE_M), lambda t, eid, act: (t, 0, 0)),
        ],
        out_specs=pl.BlockSpec((TILE_M, D), lambda t, eid, act: (t, 0)),
    )
    return pl.pallas_call(
        _ffn_body,
        grid_spec=grid_spec,
        out_shape=jax.ShapeDtypeStruct((NP, D), jnp.float32),
    )(eid, act, X_s, Wg, Wu, Wd, bd3, gates3)


def _sc_combine(O, pos0, pos1):
    O4 = O.reshape(4 * NP, QD)
    q0 = _expand_quarters(pos0).reshape(1, 4 * T)
    q1 = _expand_quarters(pos1).reshape(1, 4 * T)
    mesh = plsc.VectorSubcoreMesh(core_axis_name="core", subcore_axis_name="subcore")
    nsteps = (4 * T) // QW
    spw = nsteps // NWORKERS

    @functools.partial(
        pl.kernel,
        out_type=jax.ShapeDtypeStruct((4 * T, QD), jnp.float32),
        mesh=mesh,
        scratch_types=[pltpu.VMEM((QW, QD), jnp.float32)],
    )
    def k(o_hbm, p0_hbm, p1_hbm, out_hbm, scr):
        def body(p0_vmem, p1_vmem, out_vmem):
            pltpu.sync_copy(o_hbm.at[p0_vmem.at[0]], out_vmem)
            pltpu.sync_copy(o_hbm.at[p1_vmem.at[0]], scr)

            @pl.loop(0, QW)
            def _(r):
                for c in range(0, QD, 16):
                    slc = (pl.ds(r, 1), pl.ds(c, 16))
                    out_vmem.at[*slc][...] = out_vmem.at[*slc][...] + scr.at[*slc][...]

        pltpu.emit_pipeline(
            body,
            grid=(NWORKERS, spw),
            in_specs=[
                pl.BlockSpec((1, QW), lambda w, j, _spw=spw: (0, w * _spw + j)),
                pl.BlockSpec((1, QW), lambda w, j, _spw=spw: (0, w * _spw + j)),
            ],
            out_specs=[pl.BlockSpec((QW, QD), lambda w, j, _spw=spw: (w * _spw + j, 0))],
            core_axis_name=("core", "subcore"),
            dimension_semantics=(pltpu.PARALLEL, pltpu.ARBITRARY),
        )(p0_hbm, p1_hbm, out_hbm)

    return k(O4, q0, q1).reshape(T, D)


def kernel(x, W_router, Wg, Wu, Wd, bd):
    flat = x.reshape(T, D)
    g1, g2, i1, i2 = _router(flat, W_router)
    gidx, gates_pad, pos0, pos1, eid, act = _bookkeeping(g1, g2, i1, i2)
    X_s = _sc_gather(flat, gidx)
    O = _grouped_ffn(X_s, Wg, Wu, Wd, bd, gates_pad, eid, act)
    res = _sc_combine(O, pos0, pos1)
    return res.reshape(x.shape)
